# baseline (speedup 1.0000x reference)
import functools

import jax
import jax.numpy as jnp
from jax import lax
from jax.experimental import pallas as pl
from jax.experimental.pallas import tpu as pltpu
from jax.experimental.pallas import tpu_sc as plsc

_N = 10000
_LANES = 128
_NTILES = 32
_K = 80
_ROWS_PAD = _NTILES * _K
_K0 = 128
_K1 = 32
_ROWS_ALL = 2664
_E_PAD = _ROWS_PAD * _LANES
_NP = 10112
_STRIPE = 632
_BATCH = 1024

_mesh = plsc.VectorSubcoreMesh(core_axis_name="c", subcore_axis_name="s")


def _deg_body(dst_hbm, out_hbm, dst_v, ones_v, buf_v, acc_sh, sem):
    c = lax.axis_index("c")
    s = lax.axis_index("s")
    wid = c * 16 + s

    @pl.loop(0, 40)
    def _zero(i):
        buf_v[pl.ds(i * 16, 16)] = jnp.zeros((16,), jnp.float32)

    @pl.loop(0, 8)
    def _ones(i):
        ones_v[pl.ds(i * 16, 16)] = jnp.ones((16,), jnp.float32)

    pltpu.sync_copy(buf_v.at[pl.ds(0, _STRIPE)],
                    acc_sh.at[pl.ds(s * _STRIPE, _STRIPE)])
    plsc.subcore_barrier()
    pltpu.sync_copy(dst_hbm.at[pl.ds(wid * _K, _K)], dst_v)

    @pl.loop(0, _K)
    def _edges(j):
        pltpu.sync_copy(ones_v, acc_sh.at[dst_v.at[j]], add=True)

    plsc.subcore_barrier()
    pltpu.sync_copy(acc_sh.at[pl.ds(s * _STRIPE, _STRIPE)],
                    buf_v.at[pl.ds(0, _STRIPE)])
    pltpu.sync_copy(buf_v.at[pl.ds(0, _STRIPE)],
                    out_hbm.at[pl.ds(c * _NP + s * _STRIPE, _STRIPE)])


@functools.partial(
    pl.kernel,
    out_type=jax.ShapeDtypeStruct((2 * _NP,), jnp.float32),
    mesh=_mesh,
    scratch_types=[
        pltpu.VMEM((_K, _LANES), jnp.int32),
        pltpu.VMEM((_LANES,), jnp.float32),
        pltpu.VMEM((_STRIPE + 8,), jnp.float32),
        pltpu.VMEM_SHARED((_NP,), jnp.float32),
        pltpu.SemaphoreType.DMA,
    ],
)
def _deg_kernel(dst_hbm, out_hbm, dst_v, ones_v, buf_v, acc_sh, sem):
    _deg_body(dst_hbm, out_hbm, dst_v, ones_v, buf_v, acc_sh, sem)


def _agg_body(hs_hbm, src_hbm, dst_hbm, out_hbm, si_v, dst_v, r0_v, r1_v,
              acc_sh, g0, g1, ip0, ip1):
    c = lax.axis_index("c")
    s = lax.axis_index("s")
    base = s * _STRIPE
    kc = jnp.where(c == 0, _K1, _K0)
    row0 = c * 16 * _K1 + s * kc

    @pl.loop(0, _LANES)
    def _zero(i):
        for cl in range(8):
            r0_v[i, pl.ds(cl * 16, 16)] = jnp.zeros((16,), jnp.float32)

    for k in range(4):
        pltpu.sync_copy(r0_v, acc_sh.at[pl.ds(base + k * 128, 128)])
    pltpu.sync_copy(r0_v.at[pl.ds(0, 120)],
                    acc_sh.at[pl.ds(base + 512, 120)])
    plsc.subcore_barrier()

    pltpu.sync_copy(dst_hbm.at[pl.ds(row0, _K0)], dst_v)
    pltpu.sync_copy(src_hbm.at[pl.ds(row0, 2)], si_v)
    pltpu.async_copy(hs_hbm.at[si_v.at[0]], r0_v, g0)

    @pl.loop(0, kc // 2)
    def _edges(t):
        j = 2 * t
        pltpu.make_async_copy(hs_hbm.at[si_v.at[0]], r0_v, g0).wait()
        pltpu.async_copy(hs_hbm.at[si_v.at[1]], r1_v, g1)
        pltpu.async_copy(src_hbm.at[pl.ds(row0 + j + 2, 1)],
                         si_v.at[pl.ds(0, 1)], ip0)
        pltpu.sync_copy(r0_v, acc_sh.at[dst_v.at[j]], add=True)
        pltpu.make_async_copy(hs_hbm.at[si_v.at[1]], r1_v, g1).wait()
        pltpu.async_copy(src_hbm.at[pl.ds(row0 + j + 3, 1)],
                         si_v.at[pl.ds(1, 1)], ip1)
        pltpu.make_async_copy(src_hbm.at[pl.ds(row0 + j + 2, 1)],
                              si_v.at[pl.ds(0, 1)], ip0).wait()
        pltpu.async_copy(hs_hbm.at[si_v.at[0]], r0_v, g0)
        pltpu.sync_copy(r1_v, acc_sh.at[dst_v.at[j + 1]], add=True)
        pltpu.make_async_copy(src_hbm.at[pl.ds(row0 + j + 3, 1)],
                              si_v.at[pl.ds(1, 1)], ip1).wait()

    pltpu.make_async_copy(hs_hbm.at[si_v.at[0]], r0_v, g0).wait()

    plsc.subcore_barrier()
    for k in range(4):
        chunk = pl.ds(base + k * 128, 128)
        pltpu.sync_copy(acc_sh.at[chunk], r0_v)
        pltpu.sync_copy(r0_v, out_hbm.at[c, chunk])
    tail = pl.ds(base + 512, 120)
    pltpu.sync_copy(acc_sh.at[tail], r0_v.at[pl.ds(0, 120)])
    pltpu.sync_copy(r0_v.at[pl.ds(0, 120)], out_hbm.at[c, tail])


@functools.partial(
    pl.kernel,
    out_type=jax.ShapeDtypeStruct((2, _NP, 128), jnp.float32),
    mesh=_mesh,
    scratch_types=[
        pltpu.VMEM((2, _LANES), jnp.int32),
        pltpu.VMEM((_K0, _LANES), jnp.int32),
        pltpu.VMEM((_LANES, 128), jnp.float32),
        pltpu.VMEM((_LANES, 128), jnp.float32),
        pltpu.VMEM_SHARED((_NP, 128), jnp.float32),
        pltpu.SemaphoreType.DMA,
        pltpu.SemaphoreType.DMA,
        pltpu.SemaphoreType.DMA,
        pltpu.SemaphoreType.DMA,
    ],
)
def _agg_kernel(hs_hbm, src_hbm, dst_hbm, out_hbm, si_v, dst_v, r0_v, r1_v,
                acc_sh, g0, g1, ip0, ip1):
    _agg_body(hs_hbm, src_hbm, dst_hbm, out_hbm, si_v, dst_v, r0_v, r1_v,
              acc_sh, g0, g1, ip0, ip1)


def _mm_t(a, b):
    return lax.dot_general(a, b, (((1,), (1,)), ((), ())),
                           preferred_element_type=jnp.float32)


def _tc1_body(x_ref, w1_ref, b1_ref, wg_ref, d0_ref, d1_ref, hs_ref, dv_ref):
    deg = d0_ref[...] + d1_ref[...] + 1.0
    dv = lax.rsqrt(deg)
    h = jnp.maximum(_mm_t(x_ref[...], w1_ref[...]) + b1_ref[...], 0.0)
    hs_ref[...] = _mm_t(h, wg_ref[...]) * dv
    dv_ref[...] = dv


def _tc2_body(a0_ref, a1_ref, hs_ref, dv_ref, bg_ref, wg2_ref, out_ref):
    dv = dv_ref[...]
    h2 = dv * (a0_ref[...] + a1_ref[...] + hs_ref[...]) + bg_ref[...]
    out_ref[...] = _mm_t(h2, wg2_ref[...]) * dv


def _tc3_body(a0_ref, a1_ref, hs_ref, dv_ref, bg2_ref, w2_ref,
              b2_ref, wc_ref, bc_ref, out_ref):
    h3 = dv_ref[...] * (a0_ref[...] + a1_ref[...] + hs_ref[...]) + bg2_ref[...]
    h4 = jnp.maximum(_mm_t(h3, w2_ref[...]) + b2_ref[...], 0.0)
    out_ref[...] = _mm_t(h4, wc_ref[...]) + bc_ref[...]


_BN = 1000


def _tc1(x, W1, b1, Wg, d0, d1):
    in_dim = x.shape[1]
    return pl.pallas_call(
        _tc1_body,
        grid=(_N // _BN,),
        in_specs=[
            pl.BlockSpec((_BN, in_dim), lambda i: (i, 0)),
            pl.BlockSpec((128, in_dim), lambda i: (0, 0)),
            pl.BlockSpec((1, 128), lambda i: (0, 0)),
            pl.BlockSpec((128, 128), lambda i: (0, 0)),
            pl.BlockSpec((_BN, 1), lambda i: (i, 0)),
            pl.BlockSpec((_BN, 1), lambda i: (i, 0)),
        ],
        out_specs=[
            pl.BlockSpec((_BN, 128), lambda i: (i, 0)),
            pl.BlockSpec((_BN, 1), lambda i: (i, 0)),
        ],
        out_shape=[
            jax.ShapeDtypeStruct((_N, 128), jnp.float32),
            jax.ShapeDtypeStruct((_N, 1), jnp.float32),
        ],
    )(x, W1, b1, Wg, d0, d1)


def _tc2(a0, a1, hs, dv, bg, Wg2):
    return pl.pallas_call(
        _tc2_body,
        grid=(_N // _BN,),
        in_specs=[
            pl.BlockSpec((_BN, 128), lambda i: (i, 0)),
            pl.BlockSpec((_BN, 128), lambda i: (i, 0)),
            pl.BlockSpec((_BN, 128), lambda i: (i, 0)),
            pl.BlockSpec((_BN, 1), lambda i: (i, 0)),
            pl.BlockSpec((1, 128), lambda i: (0, 0)),
            pl.BlockSpec((128, 128), lambda i: (0, 0)),
        ],
        out_specs=pl.BlockSpec((_BN, 128), lambda i: (i, 0)),
        out_shape=jax.ShapeDtypeStruct((_N, 128), jnp.float32),
    )(a0, a1, hs, dv, bg, Wg2)


def _tc3(a0, a1, hs, dv, bg2, W2, b2, Wc, bc):
    return pl.pallas_call(
        _tc3_body,
        out_shape=jax.ShapeDtypeStruct((_BATCH, 2), jnp.float32),
    )(a0, a1, hs, dv, bg2, W2, b2, Wc, bc)


def kernel(x, edge_index, batch_size, W1, b1, Wg1, bg1, Wg2, bg2, W2, b2,
           Wc, bc):
    e = edge_index.shape[1]
    src = jnp.concatenate(
        [edge_index[0], jnp.zeros((_ROWS_ALL * _LANES - e,), jnp.int32)]
    ).reshape(_ROWS_ALL, _LANES)
    dst = jnp.concatenate(
        [edge_index[1], jnp.full((_ROWS_ALL * _LANES - e,), _N, jnp.int32)]
    ).reshape(_ROWS_ALL, _LANES)

    deg = _deg_kernel(dst).reshape(2, _NP)
    d0 = deg[0, :_N].reshape(_N, 1)
    d1 = deg[1, :_N].reshape(_N, 1)

    hs1, dinv = _tc1(x, W1, b1.reshape(1, -1), Wg1, d0, d1)

    agg1 = _agg_kernel(hs1, src, dst)
    hs2 = _tc2(agg1[0, :_N], agg1[1, :_N], hs1, dinv, bg1.reshape(1, -1), Wg2)

    agg2 = _agg_kernel(hs2, src, dst)

    start = jnp.asarray(batch_size, jnp.int32) - _BATCH
    a0s = lax.dynamic_slice_in_dim(agg2[0, :_N], start, _BATCH)
    a1s = lax.dynamic_slice_in_dim(agg2[1, :_N], start, _BATCH)
    hss = lax.dynamic_slice_in_dim(hs2, start, _BATCH)
    dvs = lax.dynamic_slice_in_dim(dinv, start, _BATCH)

    return _tc3(a0s, a1s, hss, dvs, bg2.reshape(1, -1),
                W2, b2.reshape(1, -1), Wc, bc.reshape(1, -1))


# speedup vs baseline: 1.1099x; 1.1099x over previous
import functools

import jax
import jax.numpy as jnp
from jax import lax
from jax.experimental import pallas as pl
from jax.experimental.pallas import tpu as pltpu
from jax.experimental.pallas import tpu_sc as plsc

_N = 10000
_LANES = 128
_NTILES = 32
_K = 80
_ROWS_PAD = _NTILES * _K
_K0 = 152
_K1 = 8
_ROWS_ALL = 2664
_E_PAD = _ROWS_PAD * _LANES
_NP = 10112
_STRIPE = 632
_BATCH = 1024

_mesh = plsc.VectorSubcoreMesh(core_axis_name="c", subcore_axis_name="s")


def _deg_body(dst_hbm, out_hbm, dst_v, ones_v, buf_v, acc_sh, sem):
    c = lax.axis_index("c")
    s = lax.axis_index("s")
    wid = c * 16 + s

    @pl.loop(0, 40)
    def _zero(i):
        buf_v[pl.ds(i * 16, 16)] = jnp.zeros((16,), jnp.float32)

    @pl.loop(0, 8)
    def _ones(i):
        ones_v[pl.ds(i * 16, 16)] = jnp.ones((16,), jnp.float32)

    pltpu.sync_copy(buf_v.at[pl.ds(0, _STRIPE)],
                    acc_sh.at[pl.ds(s * _STRIPE, _STRIPE)])
    plsc.subcore_barrier()
    pltpu.sync_copy(dst_hbm.at[pl.ds(wid * _K, _K)], dst_v)

    @pl.loop(0, _K)
    def _edges(j):
        pltpu.sync_copy(ones_v, acc_sh.at[dst_v.at[j]], add=True)

    plsc.subcore_barrier()
    pltpu.sync_copy(acc_sh.at[pl.ds(s * _STRIPE, _STRIPE)],
                    buf_v.at[pl.ds(0, _STRIPE)])
    pltpu.sync_copy(buf_v.at[pl.ds(0, _STRIPE)],
                    out_hbm.at[pl.ds(c * _NP + s * _STRIPE, _STRIPE)])


@functools.partial(
    pl.kernel,
    out_type=jax.ShapeDtypeStruct((2 * _NP,), jnp.float32),
    mesh=_mesh,
    scratch_types=[
        pltpu.VMEM((_K, _LANES), jnp.int32),
        pltpu.VMEM((_LANES,), jnp.float32),
        pltpu.VMEM((_STRIPE + 8,), jnp.float32),
        pltpu.VMEM_SHARED((_NP,), jnp.float32),
        pltpu.SemaphoreType.DMA,
    ],
)
def _deg_kernel(dst_hbm, out_hbm, dst_v, ones_v, buf_v, acc_sh, sem):
    _deg_body(dst_hbm, out_hbm, dst_v, ones_v, buf_v, acc_sh, sem)


def _agg_body(hs_hbm, src_hbm, dst_hbm, out_hbm, si_v, di_v, r0_v, r1_v,
              acc_sh, g0, g1, ip0, ip1, id0, id1):
    c = lax.axis_index("c")
    s = lax.axis_index("s")
    base = s * _STRIPE
    kc = jnp.where(c == 0, _K0, _K1)
    row0 = c * 16 * _K0 + s * kc

    @pl.loop(0, _LANES)
    def _zero(i):
        for cl in range(8):
            r0_v[i, pl.ds(cl * 16, 16)] = jnp.zeros((16,), jnp.float32)

    for k in range(4):
        pltpu.sync_copy(r0_v, acc_sh.at[pl.ds(base + k * 128, 128)])
    pltpu.sync_copy(r0_v.at[pl.ds(0, 120)],
                    acc_sh.at[pl.ds(base + 512, 120)])
    plsc.subcore_barrier()

    pltpu.sync_copy(src_hbm.at[pl.ds(row0, 2)], si_v)
    pltpu.sync_copy(dst_hbm.at[pl.ds(row0, 2)], di_v)
    pltpu.async_copy(hs_hbm.at[si_v.at[0]], r0_v, g0)

    @pl.loop(0, kc // 2)
    def _edges(t):
        j = 2 * t
        pltpu.make_async_copy(hs_hbm.at[si_v.at[0]], r0_v, g0).wait()
        pltpu.async_copy(hs_hbm.at[si_v.at[1]], r1_v, g1)
        pltpu.sync_copy(r0_v, acc_sh.at[di_v.at[0]], add=True)
        pltpu.async_copy(src_hbm.at[pl.ds(row0 + j + 2, 1)],
                         si_v.at[pl.ds(0, 1)], ip0)
        pltpu.async_copy(dst_hbm.at[pl.ds(row0 + j + 2, 1)],
                         di_v.at[pl.ds(0, 1)], id0)
        pltpu.make_async_copy(hs_hbm.at[si_v.at[1]], r1_v, g1).wait()
        pltpu.make_async_copy(src_hbm.at[pl.ds(row0 + j + 2, 1)],
                              si_v.at[pl.ds(0, 1)], ip0).wait()
        pltpu.async_copy(hs_hbm.at[si_v.at[0]], r0_v, g0)
        pltpu.sync_copy(r1_v, acc_sh.at[di_v.at[1]], add=True)
        pltpu.async_copy(src_hbm.at[pl.ds(row0 + j + 3, 1)],
                         si_v.at[pl.ds(1, 1)], ip1)
        pltpu.async_copy(dst_hbm.at[pl.ds(row0 + j + 3, 1)],
                         di_v.at[pl.ds(1, 1)], id1)
        pltpu.make_async_copy(dst_hbm.at[pl.ds(row0 + j + 2, 1)],
                              di_v.at[pl.ds(0, 1)], id0).wait()
        pltpu.make_async_copy(src_hbm.at[pl.ds(row0 + j + 3, 1)],
                              si_v.at[pl.ds(1, 1)], ip1).wait()
        pltpu.make_async_copy(dst_hbm.at[pl.ds(row0 + j + 3, 1)],
                              di_v.at[pl.ds(1, 1)], id1).wait()

    pltpu.make_async_copy(hs_hbm.at[si_v.at[0]], r0_v, g0).wait()

    plsc.subcore_barrier()
    for k in range(4):
        chunk = pl.ds(base + k * 128, 128)
        pltpu.sync_copy(acc_sh.at[chunk], r0_v)
        pltpu.sync_copy(r0_v, out_hbm.at[c, chunk])
    tail = pl.ds(base + 512, 120)
    pltpu.sync_copy(acc_sh.at[tail], r0_v.at[pl.ds(0, 120)])
    pltpu.sync_copy(r0_v.at[pl.ds(0, 120)], out_hbm.at[c, tail])


@functools.partial(
    pl.kernel,
    out_type=jax.ShapeDtypeStruct((2, _NP, 128), jnp.float32),
    mesh=_mesh,
    scratch_types=[
        pltpu.VMEM((2, _LANES), jnp.int32),
        pltpu.VMEM((2, _LANES), jnp.int32),
        pltpu.VMEM((_LANES, 128), jnp.float32),
        pltpu.VMEM((_LANES, 128), jnp.float32),
        pltpu.VMEM_SHARED((_NP, 128), jnp.float32),
        pltpu.SemaphoreType.DMA,
        pltpu.SemaphoreType.DMA,
        pltpu.SemaphoreType.DMA,
        pltpu.SemaphoreType.DMA,
        pltpu.SemaphoreType.DMA,
        pltpu.SemaphoreType.DMA,
    ],
)
def _agg_kernel(hs_hbm, src_hbm, dst_hbm, out_hbm, si_v, di_v, r0_v, r1_v,
                acc_sh, g0, g1, ip0, ip1, id0, id1):
    _agg_body(hs_hbm, src_hbm, dst_hbm, out_hbm, si_v, di_v, r0_v, r1_v,
              acc_sh, g0, g1, ip0, ip1, id0, id1)


def _mm_t(a, b):
    return lax.dot_general(a, b, (((1,), (1,)), ((), ())),
                           preferred_element_type=jnp.float32)


def _tc1_body(x_ref, w1_ref, b1_ref, wg_ref, d0_ref, d1_ref, hs_ref, dv_ref):
    deg = d0_ref[...] + d1_ref[...] + 1.0
    dv = lax.rsqrt(deg)
    h = jnp.maximum(_mm_t(x_ref[...], w1_ref[...]) + b1_ref[...], 0.0)
    hs_ref[...] = _mm_t(h, wg_ref[...]) * dv
    dv_ref[...] = dv


def _tc2_body(a0_ref, a1_ref, hs_ref, dv_ref, bg_ref, wg2_ref, out_ref):
    dv = dv_ref[...]
    h2 = dv * (a0_ref[...] + a1_ref[...] + hs_ref[...]) + bg_ref[...]
    out_ref[...] = _mm_t(h2, wg2_ref[...]) * dv


def _tc3_body(a0_ref, a1_ref, hs_ref, dv_ref, bg2_ref, w2_ref,
              b2_ref, wc_ref, bc_ref, out_ref):
    h3 = dv_ref[...] * (a0_ref[...] + a1_ref[...] + hs_ref[...]) + bg2_ref[...]
    h4 = jnp.maximum(_mm_t(h3, w2_ref[...]) + b2_ref[...], 0.0)
    out_ref[...] = _mm_t(h4, wc_ref[...]) + bc_ref[...]


_BN = 1000


def _tc1(x, W1, b1, Wg, d0, d1):
    in_dim = x.shape[1]
    return pl.pallas_call(
        _tc1_body,
        grid=(_N // _BN,),
        in_specs=[
            pl.BlockSpec((_BN, in_dim), lambda i: (i, 0)),
            pl.BlockSpec((128, in_dim), lambda i: (0, 0)),
            pl.BlockSpec((1, 128), lambda i: (0, 0)),
            pl.BlockSpec((128, 128), lambda i: (0, 0)),
            pl.BlockSpec((_BN, 1), lambda i: (i, 0)),
            pl.BlockSpec((_BN, 1), lambda i: (i, 0)),
        ],
        out_specs=[
            pl.BlockSpec((_BN, 128), lambda i: (i, 0)),
            pl.BlockSpec((_BN, 1), lambda i: (i, 0)),
        ],
        out_shape=[
            jax.ShapeDtypeStruct((_N, 128), jnp.float32),
            jax.ShapeDtypeStruct((_N, 1), jnp.float32),
        ],
    )(x, W1, b1, Wg, d0, d1)


def _tc2(a0, a1, hs, dv, bg, Wg2):
    return pl.pallas_call(
        _tc2_body,
        grid=(_N // _BN,),
        in_specs=[
            pl.BlockSpec((_BN, 128), lambda i: (i, 0)),
            pl.BlockSpec((_BN, 128), lambda i: (i, 0)),
            pl.BlockSpec((_BN, 128), lambda i: (i, 0)),
            pl.BlockSpec((_BN, 1), lambda i: (i, 0)),
            pl.BlockSpec((1, 128), lambda i: (0, 0)),
            pl.BlockSpec((128, 128), lambda i: (0, 0)),
        ],
        out_specs=pl.BlockSpec((_BN, 128), lambda i: (i, 0)),
        out_shape=jax.ShapeDtypeStruct((_N, 128), jnp.float32),
    )(a0, a1, hs, dv, bg, Wg2)


def _tc3(a0, a1, hs, dv, bg2, W2, b2, Wc, bc):
    return pl.pallas_call(
        _tc3_body,
        out_shape=jax.ShapeDtypeStruct((_BATCH, 2), jnp.float32),
    )(a0, a1, hs, dv, bg2, W2, b2, Wc, bc)


def kernel(x, edge_index, batch_size, W1, b1, Wg1, bg1, Wg2, bg2, W2, b2,
           Wc, bc):
    e = edge_index.shape[1]
    src = jnp.concatenate(
        [edge_index[0], jnp.zeros((_ROWS_ALL * _LANES - e,), jnp.int32)]
    ).reshape(_ROWS_ALL, _LANES)
    dst = jnp.concatenate(
        [edge_index[1], jnp.full((_ROWS_ALL * _LANES - e,), _N, jnp.int32)]
    ).reshape(_ROWS_ALL, _LANES)

    deg = _deg_kernel(dst).reshape(2, _NP)
    d0 = deg[0, :_N].reshape(_N, 1)
    d1 = deg[1, :_N].reshape(_N, 1)

    hs1, dinv = _tc1(x, W1, b1.reshape(1, -1), Wg1, d0, d1)

    agg1 = _agg_kernel(hs1, src, dst)
    hs2 = _tc2(agg1[0, :_N], agg1[1, :_N], hs1, dinv, bg1.reshape(1, -1), Wg2)

    agg2 = _agg_kernel(hs2, src, dst)

    start = jnp.asarray(batch_size, jnp.int32) - _BATCH
    a0s = lax.dynamic_slice_in_dim(agg2[0, :_N], start, _BATCH)
    a1s = lax.dynamic_slice_in_dim(agg2[1, :_N], start, _BATCH)
    hss = lax.dynamic_slice_in_dim(hs2, start, _BATCH)
    dvs = lax.dynamic_slice_in_dim(dinv, start, _BATCH)

    return _tc3(a0s, a1s, hss, dvs, bg2.reshape(1, -1),
                W2, b2.reshape(1, -1), Wc, bc.reshape(1, -1))


# speedup vs baseline: 1.1510x; 1.0370x over previous
import functools

import jax
import jax.numpy as jnp
from jax import lax
from jax.experimental import pallas as pl
from jax.experimental.pallas import tpu as pltpu
from jax.experimental.pallas import tpu_sc as plsc

_N = 10000
_LANES = 128
_NTILES = 32
_K = 80
_ROWS_PAD = _NTILES * _K
_K0 = 128
_K1 = 32
_ROWS_ALL = 2664
_E_PAD = _ROWS_PAD * _LANES
_NP = 10112
_STRIPE = 632
_BATCH = 1024

_mesh = plsc.VectorSubcoreMesh(core_axis_name="c", subcore_axis_name="s")


def _deg_body(dst_hbm, out_hbm, dst_v, ones_v, buf_v, acc_sh, sem):
    c = lax.axis_index("c")
    s = lax.axis_index("s")
    wid = c * 16 + s

    @pl.loop(0, 40)
    def _zero(i):
        buf_v[pl.ds(i * 16, 16)] = jnp.zeros((16,), jnp.float32)

    @pl.loop(0, 8)
    def _ones(i):
        ones_v[pl.ds(i * 16, 16)] = jnp.ones((16,), jnp.float32)

    pltpu.sync_copy(buf_v.at[pl.ds(0, _STRIPE)],
                    acc_sh.at[pl.ds(s * _STRIPE, _STRIPE)])
    plsc.subcore_barrier()
    pltpu.sync_copy(dst_hbm.at[pl.ds(wid * _K, _K)], dst_v)

    @pl.loop(0, _K)
    def _edges(j):
        pltpu.sync_copy(ones_v, acc_sh.at[dst_v.at[j]], add=True)

    plsc.subcore_barrier()
    pltpu.sync_copy(acc_sh.at[pl.ds(s * _STRIPE, _STRIPE)],
                    buf_v.at[pl.ds(0, _STRIPE)])
    pltpu.sync_copy(buf_v.at[pl.ds(0, _STRIPE)],
                    out_hbm.at[pl.ds(c * _NP + s * _STRIPE, _STRIPE)])


@functools.partial(
    pl.kernel,
    out_type=jax.ShapeDtypeStruct((2 * _NP,), jnp.float32),
    mesh=_mesh,
    scratch_types=[
        pltpu.VMEM((_K, _LANES), jnp.int32),
        pltpu.VMEM((_LANES,), jnp.float32),
        pltpu.VMEM((_STRIPE + 8,), jnp.float32),
        pltpu.VMEM_SHARED((_NP,), jnp.float32),
        pltpu.SemaphoreType.DMA,
    ],
)
def _deg_kernel(dst_hbm, out_hbm, dst_v, ones_v, buf_v, acc_sh, sem):
    _deg_body(dst_hbm, out_hbm, dst_v, ones_v, buf_v, acc_sh, sem)


def _agg_body(hs_hbm, src_hbm, dst_hbm, out_hbm, si_v, di_v, r0_v, r1_v,
              acc_sh, g0, g1, ip0, ip1):
    c = lax.axis_index("c")
    s = lax.axis_index("s")
    base = s * _STRIPE
    kc = jnp.where(c == 0, _K0, _K1)
    row0 = c * 16 * _K0 + s * kc

    @pl.loop(0, _LANES)
    def _zero(i):
        for cl in range(8):
            r0_v[i, pl.ds(cl * 16, 16)] = jnp.zeros((16,), jnp.float32)

    for k in range(4):
        pltpu.sync_copy(r0_v, acc_sh.at[pl.ds(base + k * 128, 128)])
    pltpu.sync_copy(r0_v.at[pl.ds(0, 120)],
                    acc_sh.at[pl.ds(base + 512, 120)])
    plsc.subcore_barrier()

    pltpu.sync_copy(dst_hbm.at[pl.ds(row0, _K0)], di_v)
    pltpu.sync_copy(src_hbm.at[pl.ds(row0, 2)], si_v)
    pltpu.async_copy(hs_hbm.at[si_v.at[0]], r0_v, g0)

    @pl.loop(0, kc // 2)
    def _edges(t):
        j = 2 * t
        pltpu.make_async_copy(hs_hbm.at[si_v.at[0]], r0_v, g0).wait()
        pltpu.async_copy(hs_hbm.at[si_v.at[1]], r1_v, g1)
        pltpu.async_copy(src_hbm.at[pl.ds(row0 + j + 2, 1)],
                         si_v.at[pl.ds(0, 1)], ip0)
        pltpu.sync_copy(r0_v, acc_sh.at[di_v.at[j]], add=True)
        pltpu.make_async_copy(hs_hbm.at[si_v.at[1]], r1_v, g1).wait()
        pltpu.async_copy(src_hbm.at[pl.ds(row0 + j + 3, 1)],
                         si_v.at[pl.ds(1, 1)], ip1)
        pltpu.make_async_copy(src_hbm.at[pl.ds(row0 + j + 2, 1)],
                              si_v.at[pl.ds(0, 1)], ip0).wait()
        pltpu.async_copy(hs_hbm.at[si_v.at[0]], r0_v, g0)
        pltpu.sync_copy(r1_v, acc_sh.at[di_v.at[j + 1]], add=True)
        pltpu.make_async_copy(src_hbm.at[pl.ds(row0 + j + 3, 1)],
                              si_v.at[pl.ds(1, 1)], ip1).wait()

    pltpu.make_async_copy(hs_hbm.at[si_v.at[0]], r0_v, g0).wait()

    plsc.subcore_barrier()
    for k in range(4):
        chunk = pl.ds(base + k * 128, 128)
        pltpu.sync_copy(acc_sh.at[chunk], r0_v)
        pltpu.sync_copy(r0_v, out_hbm.at[c, chunk])
    tail = pl.ds(base + 512, 120)
    pltpu.sync_copy(acc_sh.at[tail], r0_v.at[pl.ds(0, 120)])
    pltpu.sync_copy(r0_v.at[pl.ds(0, 120)], out_hbm.at[c, tail])


@functools.partial(
    pl.kernel,
    out_type=jax.ShapeDtypeStruct((2, _NP, 128), jnp.float32),
    mesh=_mesh,
    scratch_types=[
        pltpu.VMEM((2, _LANES), jnp.int32),
        pltpu.VMEM((_K0, _LANES), jnp.int32),
        pltpu.VMEM((_LANES, 128), jnp.float32),
        pltpu.VMEM((_LANES, 128), jnp.float32),
        pltpu.VMEM_SHARED((_NP, 128), jnp.float32),
        pltpu.SemaphoreType.DMA,
        pltpu.SemaphoreType.DMA,
        pltpu.SemaphoreType.DMA,
        pltpu.SemaphoreType.DMA,
    ],
)
def _agg_kernel(hs_hbm, src_hbm, dst_hbm, out_hbm, si_v, di_v, r0_v, r1_v,
                acc_sh, g0, g1, ip0, ip1):
    _agg_body(hs_hbm, src_hbm, dst_hbm, out_hbm, si_v, di_v, r0_v, r1_v,
              acc_sh, g0, g1, ip0, ip1)


def _mm_t(a, b):
    return lax.dot_general(a, b, (((1,), (1,)), ((), ())),
                           preferred_element_type=jnp.float32)


def _tc1_body(x_ref, w1_ref, b1_ref, wg_ref, d0_ref, d1_ref, hs_ref, dv_ref):
    deg = d0_ref[...] + d1_ref[...] + 1.0
    dv = lax.rsqrt(deg)
    h = jnp.maximum(_mm_t(x_ref[...], w1_ref[...]) + b1_ref[...], 0.0)
    hs_ref[...] = _mm_t(h, wg_ref[...]) * dv
    dv_ref[...] = dv


def _tc2_body(a0_ref, a1_ref, hs_ref, dv_ref, bg_ref, wg2_ref, out_ref):
    dv = dv_ref[...]
    h2 = dv * (a0_ref[...] + a1_ref[...] + hs_ref[...]) + bg_ref[...]
    out_ref[...] = _mm_t(h2, wg2_ref[...]) * dv


def _tc3_body(a0_ref, a1_ref, hs_ref, dv_ref, bg2_ref, w2_ref,
              b2_ref, wc_ref, bc_ref, out_ref):
    h3 = dv_ref[...] * (a0_ref[...] + a1_ref[...] + hs_ref[...]) + bg2_ref[...]
    h4 = jnp.maximum(_mm_t(h3, w2_ref[...]) + b2_ref[...], 0.0)
    out_ref[...] = _mm_t(h4, wc_ref[...]) + bc_ref[...]


_BN = 1000


def _tc1(x, W1, b1, Wg, d0, d1):
    in_dim = x.shape[1]
    return pl.pallas_call(
        _tc1_body,
        grid=(_N // _BN,),
        in_specs=[
            pl.BlockSpec((_BN, in_dim), lambda i: (i, 0)),
            pl.BlockSpec((128, in_dim), lambda i: (0, 0)),
            pl.BlockSpec((1, 128), lambda i: (0, 0)),
            pl.BlockSpec((128, 128), lambda i: (0, 0)),
            pl.BlockSpec((_BN, 1), lambda i: (i, 0)),
            pl.BlockSpec((_BN, 1), lambda i: (i, 0)),
        ],
        out_specs=[
            pl.BlockSpec((_BN, 128), lambda i: (i, 0)),
            pl.BlockSpec((_BN, 1), lambda i: (i, 0)),
        ],
        out_shape=[
            jax.ShapeDtypeStruct((_N, 128), jnp.float32),
            jax.ShapeDtypeStruct((_N, 1), jnp.float32),
        ],
    )(x, W1, b1, Wg, d0, d1)


def _tc2(a0, a1, hs, dv, bg, Wg2):
    return pl.pallas_call(
        _tc2_body,
        grid=(_N // _BN,),
        in_specs=[
            pl.BlockSpec((_BN, 128), lambda i: (i, 0)),
            pl.BlockSpec((_BN, 128), lambda i: (i, 0)),
            pl.BlockSpec((_BN, 128), lambda i: (i, 0)),
            pl.BlockSpec((_BN, 1), lambda i: (i, 0)),
            pl.BlockSpec((1, 128), lambda i: (0, 0)),
            pl.BlockSpec((128, 128), lambda i: (0, 0)),
        ],
        out_specs=pl.BlockSpec((_BN, 128), lambda i: (i, 0)),
        out_shape=jax.ShapeDtypeStruct((_N, 128), jnp.float32),
    )(a0, a1, hs, dv, bg, Wg2)


def _tc3(a0, a1, hs, dv, bg2, W2, b2, Wc, bc):
    return pl.pallas_call(
        _tc3_body,
        out_shape=jax.ShapeDtypeStruct((_BATCH, 2), jnp.float32),
    )(a0, a1, hs, dv, bg2, W2, b2, Wc, bc)


def kernel(x, edge_index, batch_size, W1, b1, Wg1, bg1, Wg2, bg2, W2, b2,
           Wc, bc):
    e = edge_index.shape[1]
    src = jnp.concatenate(
        [edge_index[0], jnp.zeros((_ROWS_ALL * _LANES - e,), jnp.int32)]
    ).reshape(_ROWS_ALL, _LANES)
    dst = jnp.concatenate(
        [edge_index[1], jnp.full((_ROWS_ALL * _LANES - e,), _N, jnp.int32)]
    ).reshape(_ROWS_ALL, _LANES)

    deg = _deg_kernel(dst).reshape(2, _NP)
    d0 = deg[0, :_N].reshape(_N, 1)
    d1 = deg[1, :_N].reshape(_N, 1)

    hs1, dinv = _tc1(x, W1, b1.reshape(1, -1), Wg1, d0, d1)

    agg1 = _agg_kernel(hs1, src, dst)
    hs2 = _tc2(agg1[0, :_N], agg1[1, :_N], hs1, dinv, bg1.reshape(1, -1), Wg2)

    agg2 = _agg_kernel(hs2, src, dst)

    start = jnp.asarray(batch_size, jnp.int32) - _BATCH
    a0s = lax.dynamic_slice_in_dim(agg2[0, :_N], start, _BATCH)
    a1s = lax.dynamic_slice_in_dim(agg2[1, :_N], start, _BATCH)
    hss = lax.dynamic_slice_in_dim(hs2, start, _BATCH)
    dvs = lax.dynamic_slice_in_dim(dinv, start, _BATCH)

    return _tc3(a0s, a1s, hss, dvs, bg2.reshape(1, -1),
                W2, b2.reshape(1, -1), Wc, bc.reshape(1, -1))


# speedup vs baseline: 1.1520x; 1.0008x over previous
import functools

import jax
import jax.numpy as jnp
from jax import lax
from jax.experimental import pallas as pl
from jax.experimental.pallas import tpu as pltpu
from jax.experimental.pallas import tpu_sc as plsc

_N = 10000
_LANES = 128
_NTILES = 32
_K = 80
_ROWS_PAD = _NTILES * _K
_K0 = 128
_K1 = 32
_ROWS_ALL = 2664
_E_PAD = _ROWS_PAD * _LANES
_NP = 10112
_STRIPE = 632
_BATCH = 1024

_mesh = plsc.VectorSubcoreMesh(core_axis_name="c", subcore_axis_name="s")


def _deg_body(dst_hbm, out_hbm, dst_v, ones_v, buf_v, acc_sh, sem):
    c = lax.axis_index("c")
    s = lax.axis_index("s")
    wid = c * 16 + s

    @pl.loop(0, 40)
    def _zero(i):
        buf_v[pl.ds(i * 16, 16)] = jnp.zeros((16,), jnp.float32)

    @pl.loop(0, 8)
    def _ones(i):
        ones_v[pl.ds(i * 16, 16)] = jnp.ones((16,), jnp.float32)

    pltpu.sync_copy(buf_v.at[pl.ds(0, _STRIPE)],
                    acc_sh.at[pl.ds(s * _STRIPE, _STRIPE)])
    plsc.subcore_barrier()
    pltpu.sync_copy(dst_hbm.at[pl.ds(wid * _K, _K)], dst_v)

    @pl.loop(0, _K)
    def _edges(j):
        pltpu.sync_copy(ones_v, acc_sh.at[dst_v.at[j]], add=True)

    plsc.subcore_barrier()
    pltpu.sync_copy(acc_sh.at[pl.ds(s * _STRIPE, _STRIPE)],
                    buf_v.at[pl.ds(0, _STRIPE)])
    pltpu.sync_copy(buf_v.at[pl.ds(0, _STRIPE)],
                    out_hbm.at[pl.ds(c * _NP + s * _STRIPE, _STRIPE)])


@functools.partial(
    pl.kernel,
    out_type=jax.ShapeDtypeStruct((2 * _NP,), jnp.float32),
    mesh=_mesh,
    scratch_types=[
        pltpu.VMEM((_K, _LANES), jnp.int32),
        pltpu.VMEM((_LANES,), jnp.float32),
        pltpu.VMEM((_STRIPE + 8,), jnp.float32),
        pltpu.VMEM_SHARED((_NP,), jnp.float32),
        pltpu.SemaphoreType.DMA,
    ],
)
def _deg_kernel(dst_hbm, out_hbm, dst_v, ones_v, buf_v, acc_sh, sem):
    _deg_body(dst_hbm, out_hbm, dst_v, ones_v, buf_v, acc_sh, sem)


def _agg_body(hs_hbm, src_hbm, dst_hbm, out_hbm, si_v, di_v, r0_v, r1_v,
              acc_sh, g0, g1, ip0, ip1, s0, s1):
    c = lax.axis_index("c")
    s = lax.axis_index("s")
    base = s * _STRIPE
    kc = jnp.where(c == 0, _K0, _K1)
    row0 = c * 16 * _K0 + s * kc

    @pl.loop(0, _LANES)
    def _zero(i):
        for cl in range(8):
            r0_v[i, pl.ds(cl * 16, 16)] = jnp.zeros((16,), jnp.float32)

    for k in range(4):
        pltpu.sync_copy(r0_v, acc_sh.at[pl.ds(base + k * 128, 128)])
    pltpu.sync_copy(r0_v.at[pl.ds(0, 120)],
                    acc_sh.at[pl.ds(base + 512, 120)])
    plsc.subcore_barrier()

    pltpu.sync_copy(dst_hbm.at[pl.ds(row0, _K0)], di_v)
    pltpu.sync_copy(src_hbm.at[pl.ds(row0, 2)], si_v)
    pltpu.async_copy(hs_hbm.at[si_v.at[0]], r0_v, g0)

    @pl.loop(0, kc // 2)
    def _edges(t):
        j = 2 * t
        pltpu.make_async_copy(hs_hbm.at[si_v.at[0]], r0_v, g0).wait()
        pltpu.async_copy(hs_hbm.at[si_v.at[1]], r1_v, g1)
        pltpu.async_copy(src_hbm.at[pl.ds(row0 + j + 2, 1)],
                         si_v.at[pl.ds(0, 1)], ip0)
        pltpu.async_copy(r0_v, acc_sh.at[di_v.at[j]], s0, add=True)
        pltpu.make_async_copy(hs_hbm.at[si_v.at[1]], r1_v, g1).wait()
        pltpu.async_copy(r1_v, acc_sh.at[di_v.at[j + 1]], s1, add=True)
        pltpu.make_async_copy(src_hbm.at[pl.ds(row0 + j + 2, 1)],
                              si_v.at[pl.ds(0, 1)], ip0).wait()
        pltpu.make_async_copy(r0_v, acc_sh.at[di_v.at[j]], s0).wait()
        pltpu.async_copy(hs_hbm.at[si_v.at[0]], r0_v, g0)
        pltpu.async_copy(src_hbm.at[pl.ds(row0 + j + 3, 1)],
                         si_v.at[pl.ds(1, 1)], ip1)
        pltpu.make_async_copy(r1_v, acc_sh.at[di_v.at[j + 1]], s1).wait()
        pltpu.make_async_copy(src_hbm.at[pl.ds(row0 + j + 3, 1)],
                              si_v.at[pl.ds(1, 1)], ip1).wait()

    pltpu.make_async_copy(hs_hbm.at[si_v.at[0]], r0_v, g0).wait()

    plsc.subcore_barrier()
    for k in range(4):
        chunk = pl.ds(base + k * 128, 128)
        pltpu.sync_copy(acc_sh.at[chunk], r0_v)
        pltpu.sync_copy(r0_v, out_hbm.at[c, chunk])
    tail = pl.ds(base + 512, 120)
    pltpu.sync_copy(acc_sh.at[tail], r0_v.at[pl.ds(0, 120)])
    pltpu.sync_copy(r0_v.at[pl.ds(0, 120)], out_hbm.at[c, tail])


@functools.partial(
    pl.kernel,
    out_type=jax.ShapeDtypeStruct((2, _NP, 128), jnp.float32),
    mesh=_mesh,
    scratch_types=[
        pltpu.VMEM((2, _LANES), jnp.int32),
        pltpu.VMEM((_K0, _LANES), jnp.int32),
        pltpu.VMEM((_LANES, 128), jnp.float32),
        pltpu.VMEM((_LANES, 128), jnp.float32),
        pltpu.VMEM_SHARED((_NP, 128), jnp.float32),
        pltpu.SemaphoreType.DMA,
        pltpu.SemaphoreType.DMA,
        pltpu.SemaphoreType.DMA,
        pltpu.SemaphoreType.DMA,
        pltpu.SemaphoreType.DMA,
        pltpu.SemaphoreType.DMA,
    ],
)
def _agg_kernel(hs_hbm, src_hbm, dst_hbm, out_hbm, si_v, di_v, r0_v, r1_v,
                acc_sh, g0, g1, ip0, ip1, s0, s1):
    _agg_body(hs_hbm, src_hbm, dst_hbm, out_hbm, si_v, di_v, r0_v, r1_v,
              acc_sh, g0, g1, ip0, ip1, s0, s1)


def _mm_t(a, b):
    return lax.dot_general(a, b, (((1,), (1,)), ((), ())),
                           preferred_element_type=jnp.float32)


def _tc1_body(x_ref, w1_ref, b1_ref, wg_ref, d0_ref, d1_ref, hs_ref, dv_ref):
    deg = d0_ref[...] + d1_ref[...] + 1.0
    dv = lax.rsqrt(deg)
    h = jnp.maximum(_mm_t(x_ref[...], w1_ref[...]) + b1_ref[...], 0.0)
    hs_ref[...] = _mm_t(h, wg_ref[...]) * dv
    dv_ref[...] = dv


def _tc2_body(a0_ref, a1_ref, hs_ref, dv_ref, bg_ref, wg2_ref, out_ref):
    dv = dv_ref[...]
    h2 = dv * (a0_ref[...] + a1_ref[...] + hs_ref[...]) + bg_ref[...]
    out_ref[...] = _mm_t(h2, wg2_ref[...]) * dv


def _tc3_body(a0_ref, a1_ref, hs_ref, dv_ref, bg2_ref, w2_ref,
              b2_ref, wc_ref, bc_ref, out_ref):
    h3 = dv_ref[...] * (a0_ref[...] + a1_ref[...] + hs_ref[...]) + bg2_ref[...]
    h4 = jnp.maximum(_mm_t(h3, w2_ref[...]) + b2_ref[...], 0.0)
    out_ref[...] = _mm_t(h4, wc_ref[...]) + bc_ref[...]


_BN = 1000


def _tc1(x, W1, b1, Wg, d0, d1):
    in_dim = x.shape[1]
    return pl.pallas_call(
        _tc1_body,
        grid=(_N // _BN,),
        in_specs=[
            pl.BlockSpec((_BN, in_dim), lambda i: (i, 0)),
            pl.BlockSpec((128, in_dim), lambda i: (0, 0)),
            pl.BlockSpec((1, 128), lambda i: (0, 0)),
            pl.BlockSpec((128, 128), lambda i: (0, 0)),
            pl.BlockSpec((_BN, 1), lambda i: (i, 0)),
            pl.BlockSpec((_BN, 1), lambda i: (i, 0)),
        ],
        out_specs=[
            pl.BlockSpec((_BN, 128), lambda i: (i, 0)),
            pl.BlockSpec((_BN, 1), lambda i: (i, 0)),
        ],
        out_shape=[
            jax.ShapeDtypeStruct((_N, 128), jnp.float32),
            jax.ShapeDtypeStruct((_N, 1), jnp.float32),
        ],
    )(x, W1, b1, Wg, d0, d1)


def _tc2(a0, a1, hs, dv, bg, Wg2):
    return pl.pallas_call(
        _tc2_body,
        grid=(_N // _BN,),
        in_specs=[
            pl.BlockSpec((_BN, 128), lambda i: (i, 0)),
            pl.BlockSpec((_BN, 128), lambda i: (i, 0)),
            pl.BlockSpec((_BN, 128), lambda i: (i, 0)),
            pl.BlockSpec((_BN, 1), lambda i: (i, 0)),
            pl.BlockSpec((1, 128), lambda i: (0, 0)),
            pl.BlockSpec((128, 128), lambda i: (0, 0)),
        ],
        out_specs=pl.BlockSpec((_BN, 128), lambda i: (i, 0)),
        out_shape=jax.ShapeDtypeStruct((_N, 128), jnp.float32),
    )(a0, a1, hs, dv, bg, Wg2)


def _tc3(a0, a1, hs, dv, bg2, W2, b2, Wc, bc):
    return pl.pallas_call(
        _tc3_body,
        out_shape=jax.ShapeDtypeStruct((_BATCH, 2), jnp.float32),
    )(a0, a1, hs, dv, bg2, W2, b2, Wc, bc)


def kernel(x, edge_index, batch_size, W1, b1, Wg1, bg1, Wg2, bg2, W2, b2,
           Wc, bc):
    e = edge_index.shape[1]
    src = jnp.concatenate(
        [edge_index[0], jnp.zeros((_ROWS_ALL * _LANES - e,), jnp.int32)]
    ).reshape(_ROWS_ALL, _LANES)
    dst = jnp.concatenate(
        [edge_index[1], jnp.full((_ROWS_ALL * _LANES - e,), _N, jnp.int32)]
    ).reshape(_ROWS_ALL, _LANES)

    deg = _deg_kernel(dst).reshape(2, _NP)
    d0 = deg[0, :_N].reshape(_N, 1)
    d1 = deg[1, :_N].reshape(_N, 1)

    hs1, dinv = _tc1(x, W1, b1.reshape(1, -1), Wg1, d0, d1)

    agg1 = _agg_kernel(hs1, src, dst)
    hs2 = _tc2(agg1[0, :_N], agg1[1, :_N], hs1, dinv, bg1.reshape(1, -1), Wg2)

    agg2 = _agg_kernel(hs2, src, dst)

    start = jnp.asarray(batch_size, jnp.int32) - _BATCH
    a0s = lax.dynamic_slice_in_dim(agg2[0, :_N], start, _BATCH)
    a1s = lax.dynamic_slice_in_dim(agg2[1, :_N], start, _BATCH)
    hss = lax.dynamic_slice_in_dim(hs2, start, _BATCH)
    dvs = lax.dynamic_slice_in_dim(dinv, start, _BATCH)

    return _tc3(a0s, a1s, hss, dvs, bg2.reshape(1, -1),
                W2, b2.reshape(1, -1), Wc, bc.reshape(1, -1))
